# Initial kernel scaffold; baseline (speedup 1.0000x reference)
import functools

import jax
import jax.numpy as jnp
from jax import lax
from jax.experimental import pallas as pl
from jax.experimental.pallas import tpu as pltpu
from jax.experimental.pallas import tpu_sc as plsc

B = 4096
F = 26
V = 1000
D = 128
NV = 8
LPAD = 32
CH = 4
RPC = CH * F

_info = plsc.get_sparse_core_info()
NC, NS, L = _info.num_cores, _info.num_subcores, _info.num_lanes
NW = NC * NS
BPW = B // NW
NCH = BPW // CH
ROWS_PW = BPW * F

_mesh = plsc.VectorSubcoreMesh(core_axis_name="c", subcore_axis_name="s")


@functools.partial(
    pl.kernel,
    mesh=_mesh,
    out_type=jax.ShapeDtypeStruct((B,), jnp.float32),
    scratch_types=[
        pltpu.VMEM((F * V,), jnp.float32),
        pltpu.VMEM((ROWS_PW,), jnp.int32),
        pltpu.VMEM((BPW, LPAD), jnp.int32),
        pltpu.VMEM((RPC, D), jnp.float32),
        pltpu.VMEM((RPC, D), jnp.float32),
        pltpu.VMEM((16,), jnp.float32),
        pltpu.VMEM((BPW,), jnp.float32),
        pltpu.SemaphoreType.DMA,
        pltpu.SemaphoreType.DMA,
    ],
)
def _fcfm_sc(emb_hbm, eidx_hbm, lidx_hbm, lin_hbm, bias_hbm, out_hbm,
             lin_v, eidx_v, lidx_v, rows0, rows1, bias_v, pre_v,
             sem0, sem1):
    wid = lax.axis_index("s") * NC + lax.axis_index("c")
    row_base = wid * ROWS_PW
    samp_base = wid * BPW

    pltpu.sync_copy(lin_hbm, lin_v)
    pltpu.sync_copy(eidx_hbm.at[pl.ds(row_base, ROWS_PW)], eidx_v)
    pltpu.sync_copy(lidx_hbm.at[pl.ds(samp_base, BPW)], lidx_v)
    pltpu.sync_copy(bias_hbm, bias_v)

    def start_gather(c, buf, sem):
        pltpu.make_async_copy(
            emb_hbm.at[eidx_v.at[pl.ds(c * RPC, RPC)]], buf, sem).start()

    def wait_gather(c, buf, sem):
        pltpu.make_async_copy(
            emb_hbm.at[eidx_v.at[pl.ds(c * RPC, RPC)]], buf, sem).wait()

    lanes = lax.iota(jnp.int32, L)
    zero = jnp.zeros((L,), jnp.float32)

    def compute_chunk(c, buf):
        for ss in range(CH):
            r0 = ss * F

            def fbody(f, acc):
                accs, accq = acc
                ns, nq = [], []
                for v in range(NV):
                    r = buf[r0 + f, pl.ds(v * L, L)]
                    ns.append(accs[v] + r)
                    nq.append(accq[v] + r * r)
                return (tuple(ns), tuple(nq))

            accs, accq = lax.fori_loop(
                0, F, fbody,
                (tuple(zero for _ in range(NV)),
                 tuple(zero for _ in range(NV))))
            t = accs[0] * accs[0] - accq[0]
            for v in range(1, NV):
                t = t + (accs[v] * accs[v] - accq[v])
            emb_sum = jnp.sum(t)

            s_local = c * CH + ss
            li0 = lidx_v[s_local, pl.ds(0, L)]
            li1 = lidx_v[s_local, pl.ds(L, L)]
            g0 = plsc.load_gather(lin_v, [li0])
            g1 = plsc.load_gather(lin_v, [li1])
            g1 = jnp.where(lanes < (F - L), g1, 0.0)
            lin_sum = jnp.sum(g0 + g1)

            pre_v[s_local] = 0.5 * emb_sum + lin_sum

    start_gather(0, rows0, sem0)

    def chunk_pair(jj, _):
        j0 = 2 * jj
        start_gather(j0 + 1, rows1, sem1)
        wait_gather(j0, rows0, sem0)
        compute_chunk(j0, rows0)

        @pl.when(j0 + 2 < NCH)
        def _():
            start_gather(j0 + 2, rows0, sem0)

        wait_gather(j0 + 1, rows1, sem1)
        compute_chunk(j0 + 1, rows1)
        return 0

    lax.fori_loop(0, NCH // 2, chunk_pair, 0)

    bias_vec = bias_v[pl.ds(0, L)]
    for v8 in range(BPW // L):
        x = pre_v[pl.ds(v8 * L, L)]
        pre_v[pl.ds(v8 * L, L)] = 1.0 / (1.0 + jnp.exp(-(x + bias_vec)))

    pltpu.sync_copy(pre_v, out_hbm.at[pl.ds(samp_base, BPW)])


def kernel(indices, linear_tables, embed_tables, bias):
    idx32 = indices.astype(jnp.int32)
    gidx = idx32 + (jnp.arange(F, dtype=jnp.int32) * V)[None, :]
    eidx = gidx.reshape(B * F)
    lidx = jnp.concatenate(
        [gidx, jnp.zeros((B, LPAD - F), jnp.int32)], axis=1)
    emb_flat = embed_tables.reshape(F * V, D)
    lin_flat = linear_tables.reshape(F * V)
    bias16 = jnp.broadcast_to(bias, (L,))
    out = _fcfm_sc(emb_flat, eidx, lidx, lin_flat, bias16)
    return out.reshape(B, 1)


# speedup vs baseline: 17.9703x; 17.9703x over previous
import functools

import jax
import jax.numpy as jnp
from jax import lax
from jax.experimental import pallas as pl
from jax.experimental.pallas import tpu as pltpu
from jax.experimental.pallas import tpu_sc as plsc

B = 4096
F = 26
V = 1000
D = 128
NV = 8
LPAD = 32
CH = 4
RPC = CH * F
LPC = CH * LPAD

_info = plsc.get_sparse_core_info()
NC, NS, L = _info.num_cores, _info.num_subcores, _info.num_lanes
NW = NC * NS
BPW = B // NW
NCH = BPW // CH
ROWS_PW = BPW * F

_mesh = plsc.VectorSubcoreMesh(core_axis_name="c", subcore_axis_name="s")


@functools.partial(
    pl.kernel,
    mesh=_mesh,
    out_type=jax.ShapeDtypeStruct((B,), jnp.float32),
    scratch_types=[
        pltpu.VMEM((ROWS_PW,), jnp.int32),
        pltpu.VMEM((BPW * LPAD,), jnp.int32),
        pltpu.VMEM((RPC, D), jnp.float32),
        pltpu.VMEM((RPC, D), jnp.float32),
        pltpu.VMEM((LPC,), jnp.float32),
        pltpu.VMEM((LPC,), jnp.float32),
        pltpu.VMEM((16,), jnp.float32),
        pltpu.VMEM((BPW,), jnp.float32),
        pltpu.SemaphoreType.DMA,
        pltpu.SemaphoreType.DMA,
    ],
)
def _fcfm_sc(emb_hbm, eidx_hbm, lidx_hbm, lin_hbm, bias_hbm, out_hbm,
             eidx_v, lidx_v, rows0, rows1, lrow0, lrow1, bias_v, pre_v,
             sem0, sem1):
    wid = lax.axis_index("s") * NC + lax.axis_index("c")
    row_base = wid * ROWS_PW
    samp_base = wid * BPW

    pltpu.sync_copy(eidx_hbm.at[pl.ds(row_base, ROWS_PW)], eidx_v)
    pltpu.sync_copy(lidx_hbm.at[pl.ds(samp_base * LPAD, BPW * LPAD)], lidx_v)
    pltpu.sync_copy(bias_hbm, bias_v)

    def start_gather(c, buf, lbuf, sem):
        pltpu.make_async_copy(
            emb_hbm.at[eidx_v.at[pl.ds(c * RPC, RPC)]], buf, sem).start()
        pltpu.make_async_copy(
            lin_hbm.at[lidx_v.at[pl.ds(c * LPC, LPC)]], lbuf, sem).start()

    def wait_gather(c, buf, lbuf, sem):
        pltpu.make_async_copy(
            emb_hbm.at[eidx_v.at[pl.ds(c * RPC, RPC)]], buf, sem).wait()
        pltpu.make_async_copy(
            lin_hbm.at[lidx_v.at[pl.ds(c * LPC, LPC)]], lbuf, sem).wait()

    lanes = lax.iota(jnp.int32, L)
    zero = jnp.zeros((L,), jnp.float32)
    shuf = [(lanes + sh) & (L - 1) for sh in (8, 4, 2, 1)]

    def lane_sum(u):
        for idx in shuf:
            u = u + u.at[idx].get(mode="promise_in_bounds")
        return u

    def compute_chunk(c, buf, lbuf):
        contrib = zero
        for ss in range(CH):
            r0 = ss * F

            def fbody(f, acc):
                accs, accq = acc
                ns, nq = [], []
                for v in range(NV):
                    r = buf[r0 + f, pl.ds(v * L, L)]
                    ns.append(accs[v] + r)
                    nq.append(accq[v] + r * r)
                return (tuple(ns), tuple(nq))

            accs, accq = lax.fori_loop(
                0, F, fbody,
                (tuple(zero for _ in range(NV)),
                 tuple(zero for _ in range(NV))))
            t = accs[0] * accs[0] - accq[0]
            for v in range(1, NV):
                t = t + (accs[v] * accs[v] - accq[v])

            g0 = lbuf[pl.ds(ss * LPAD, L)]
            g1 = lbuf[pl.ds(ss * LPAD + L, L)]
            g1 = jnp.where(lanes < (F - L), g1, 0.0)

            pre = lane_sum(0.5 * t + g0 + g1)
            s_local = c * CH + ss
            contrib = contrib + jnp.where(
                lanes == lax.rem(s_local, L), pre, zero)

        win = (c // (L // CH)) * L
        plsc.addupdate(pre_v.at[pl.ds(win, L)], contrib)

    for v8 in range(BPW // L):
        pre_v[pl.ds(v8 * L, L)] = zero

    start_gather(0, rows0, lrow0, sem0)

    def chunk_pair(jj, _):
        j0 = 2 * jj
        start_gather(j0 + 1, rows1, lrow1, sem1)
        wait_gather(j0, rows0, lrow0, sem0)
        compute_chunk(j0, rows0, lrow0)

        @pl.when(j0 + 2 < NCH)
        def _():
            start_gather(j0 + 2, rows0, lrow0, sem0)

        wait_gather(j0 + 1, rows1, lrow1, sem1)
        compute_chunk(j0 + 1, rows1, lrow1)
        return 0

    lax.fori_loop(0, NCH // 2, chunk_pair, 0)

    bias_vec = bias_v[pl.ds(0, L)]
    for v8 in range(BPW // L):
        x = pre_v[pl.ds(v8 * L, L)]
        pre_v[pl.ds(v8 * L, L)] = 1.0 / (1.0 + jnp.exp(-(x + bias_vec)))

    pltpu.sync_copy(pre_v, out_hbm.at[pl.ds(samp_base, BPW)])


def kernel(indices, linear_tables, embed_tables, bias):
    idx32 = indices.astype(jnp.int32)
    gidx = idx32 + (jnp.arange(F, dtype=jnp.int32) * V)[None, :]
    eidx = gidx.reshape(B * F)
    lidx = jnp.concatenate(
        [gidx, jnp.zeros((B, LPAD - F), jnp.int32)], axis=1).reshape(B * LPAD)
    emb_flat = embed_tables.reshape(F * V, D)
    lin_flat = linear_tables.reshape(F * V)
    bias16 = jnp.broadcast_to(bias, (L,))
    out = _fcfm_sc(emb_flat, eidx, lidx, lin_flat, bias16)
    return out.reshape(B, 1)
